# baseline (device time: 131373 ns/iter reference)
import jax
import jax.numpy as jnp
from jax import lax
from jax.experimental import pallas as pl
from jax.experimental.pallas import tpu as pltpu

BH = 16
SEQ = 512
D = 64
SCALE = D ** -0.5


def kernel(Q, K, V):
    def to_bhsd(t):
        return t.transpose(0, 2, 1, 3).reshape(BH, SEQ, D)

    Qt = to_bhsd(Q)
    KV = jnp.stack([to_bhsd(K), to_bhsd(V)])

    def body(q_ref, kv_ref, out_ref, peer_ref, send_sems, recv_sems):
        my_x = lax.axis_index("x")
        my_y = lax.axis_index("y")
        y_nbr = (my_x, 1 - my_y)
        x_nbr = (1 - my_x, my_y)

        barrier_sem = pltpu.get_barrier_semaphore()
        for nbr in (y_nbr, x_nbr):
            pl.semaphore_signal(
                barrier_sem, inc=1,
                device_id=nbr, device_id_type=pl.DeviceIdType.MESH,
            )
        pl.semaphore_wait(barrier_sem, 2)

        rdma1 = pltpu.make_async_remote_copy(
            src_ref=kv_ref.at[my_x],
            dst_ref=peer_ref.at[my_x],
            send_sem=send_sems.at[0],
            recv_sem=recv_sems.at[0],
            device_id=y_nbr,
            device_id_type=pl.DeviceIdType.MESH,
        )
        rdma1.start()
        rdma1.wait()

        rdma2 = pltpu.make_async_remote_copy(
            src_ref=peer_ref.at[my_x],
            dst_ref=peer_ref.at[my_x],
            send_sem=send_sems.at[1],
            recv_sem=recv_sems.at[1],
            device_id=x_nbr,
            device_id_type=pl.DeviceIdType.MESH,
        )
        rdma2.start()
        rdma2.wait()

        def compute(i, _):
            q = q_ref[i]
            k_full = jnp.concatenate([kv_ref[0, i], peer_ref[0, i]], axis=0)
            v_full = jnp.concatenate([kv_ref[1, i], peer_ref[1, i]], axis=0)
            s = lax.dot_general(
                q, k_full, (((1,), (1,)), ((), ())),
                preferred_element_type=jnp.float32,
            ) * SCALE
            m = jnp.max(s, axis=-1, keepdims=True)
            p = jnp.exp(s - m)
            p = p / jnp.sum(p, axis=-1, keepdims=True)
            out_ref[i] = jnp.dot(p, v_full, preferred_element_type=jnp.float32)
            return 0

        lax.fori_loop(0, BH, compute, 0)

    out = pl.pallas_call(
        body,
        out_shape=jax.ShapeDtypeStruct((BH, SEQ, D), jnp.float32),
        in_specs=[
            pl.BlockSpec(memory_space=pltpu.VMEM),
            pl.BlockSpec(memory_space=pltpu.VMEM),
        ],
        out_specs=pl.BlockSpec(memory_space=pltpu.VMEM),
        scratch_shapes=[
            pltpu.VMEM((2, BH, SEQ, D), jnp.float32),
            pltpu.SemaphoreType.DMA((2,)),
            pltpu.SemaphoreType.DMA((2,)),
        ],
        compiler_params=pltpu.CompilerParams(collective_id=0),
    )(Qt, KV)

    return out.reshape(2, 8, SEQ, D).transpose(0, 2, 1, 3)


# device time: 70901 ns/iter; 1.8529x vs baseline; 1.8529x over previous
import jax
import jax.numpy as jnp
from jax import lax
from jax.experimental import pallas as pl
from jax.experimental.pallas import tpu as pltpu

BH = 16
SEQ = 512
D = 64
SCALE = D ** -0.5
C = 8
B = BH // C


def kernel(Q, K, V):
    def to_bhsd(t):
        return t.transpose(0, 2, 1, 3).reshape(BH, SEQ, D)

    Qt = to_bhsd(Q)
    KV = jnp.stack([to_bhsd(K), to_bhsd(V)])

    def body(q_ref, kv_ref, out_ref, peer_ref, den_ref, s1, r1, s2, r2):
        my_x = lax.axis_index("x")
        my_y = lax.axis_index("y")
        y_nbr = (my_x, 1 - my_y)
        x_nbr = (1 - my_x, my_y)

        barrier_sem = pltpu.get_barrier_semaphore()
        for nbr in (y_nbr, x_nbr):
            pl.semaphore_signal(
                barrier_sem, inc=1,
                device_id=nbr, device_id_type=pl.DeviceIdType.MESH,
            )
        pl.semaphore_wait(barrier_sem, 2)

        rdma1 = []
        for c in range(C):
            d = pltpu.make_async_remote_copy(
                src_ref=kv_ref.at[my_x, pl.ds(c * B, B)],
                dst_ref=peer_ref.at[my_x, pl.ds(c * B, B)],
                send_sem=s1.at[c],
                recv_sem=r1.at[c],
                device_id=y_nbr,
                device_id_type=pl.DeviceIdType.MESH,
            )
            d.start()
            rdma1.append(d)

        def local_compute(i):
            qs = q_ref[i] * SCALE
            e = jnp.exp(lax.dot_general(
                qs, kv_ref[0, i], (((1,), (1,)), ((), ())),
                preferred_element_type=jnp.float32,
            ))
            den_ref[i] = jnp.sum(e, axis=-1, keepdims=True)
            out_ref[i] = jnp.dot(
                e, kv_ref[1, i], preferred_element_type=jnp.float32
            )

        def peer_compute(i):
            qs = q_ref[i] * SCALE
            e = jnp.exp(lax.dot_general(
                qs, peer_ref[0, i], (((1,), (1,)), ((), ())),
                preferred_element_type=jnp.float32,
            ))
            den = den_ref[i] + jnp.sum(e, axis=-1, keepdims=True)
            acc = out_ref[i] + jnp.dot(
                e, peer_ref[1, i], preferred_element_type=jnp.float32
            )
            out_ref[i] = acc * (1.0 / den)

        rdma2 = []
        for c in range(C):
            rdma1[c].wait_recv()
            d = pltpu.make_async_remote_copy(
                src_ref=peer_ref.at[my_x, pl.ds(c * B, B)],
                dst_ref=peer_ref.at[my_x, pl.ds(c * B, B)],
                send_sem=s2.at[c],
                recv_sem=r2.at[c],
                device_id=x_nbr,
                device_id_type=pl.DeviceIdType.MESH,
            )
            d.start()
            rdma2.append(d)
            for i in range(c * B, (c + 1) * B):
                local_compute(i)

        for c in range(C):
            rdma2[c].wait_recv()
            for i in range(c * B, (c + 1) * B):
                peer_compute(i)

        for c in range(C):
            rdma1[c].wait_send()
            rdma2[c].wait_send()

    out = pl.pallas_call(
        body,
        out_shape=jax.ShapeDtypeStruct((BH, SEQ, D), jnp.float32),
        in_specs=[
            pl.BlockSpec(memory_space=pltpu.VMEM),
            pl.BlockSpec(memory_space=pltpu.VMEM),
        ],
        out_specs=pl.BlockSpec(memory_space=pltpu.VMEM),
        scratch_shapes=[
            pltpu.VMEM((2, BH, SEQ, D), jnp.float32),
            pltpu.VMEM((BH, SEQ, 1), jnp.float32),
            pltpu.SemaphoreType.DMA((C,)),
            pltpu.SemaphoreType.DMA((C,)),
            pltpu.SemaphoreType.DMA((C,)),
            pltpu.SemaphoreType.DMA((C,)),
        ],
        compiler_params=pltpu.CompilerParams(collective_id=0),
    )(Qt, KV)

    return out.reshape(2, 8, SEQ, D).transpose(0, 2, 1, 3)


# device time: 41033 ns/iter; 3.2016x vs baseline; 1.7279x over previous
import jax
import jax.numpy as jnp
from jax import lax
from jax.experimental import pallas as pl
from jax.experimental.pallas import tpu as pltpu

BH = 16
HALF = 8
SEQ = 512
D = 64
SCALE = D ** -0.5
LOG2E = 1.4426950408889634
C = 8
B = HALF // C


def kernel(Q, K, V):
    def to_bhsd(t):
        return t.transpose(0, 2, 1, 3).reshape(BH, SEQ, D)

    Qt = to_bhsd(Q * (SCALE * LOG2E)).astype(jnp.bfloat16)
    Kb = to_bhsd(K).astype(jnp.bfloat16)
    Vb = to_bhsd(V).astype(jnp.bfloat16)

    def body(q_ref, k_ref, v_ref, out_ref,
             kv8send_ref, scsend_ref, pkv8_ref, psc_ref,
             acc_ref, myout_ref, outx_ref,
             s1, r1, s1s, r1s, s3, r3):
        my_x = lax.axis_index("x")
        my_y = lax.axis_index("y")
        y_nbr = (my_x, 1 - my_y)
        x_nbr = (1 - my_x, my_y)
        base = my_x * HALF

        barrier_sem = pltpu.get_barrier_semaphore()
        for nbr in (y_nbr, x_nbr):
            pl.semaphore_signal(
                barrier_sem, inc=1,
                device_id=nbr, device_id_type=pl.DeviceIdType.MESH,
            )

        for j in range(HALF):
            kf = k_ref[base + j].astype(jnp.float32)
            vf = v_ref[base + j].astype(jnp.float32)
            kmax = jnp.maximum(
                jnp.max(jnp.abs(kf), axis=-1, keepdims=True), 1e-6
            )
            vmax = jnp.maximum(
                jnp.max(jnp.abs(vf), axis=-1, keepdims=True), 1e-6
            )
            kv8send_ref[j] = jnp.concatenate(
                [jnp.rint(kf * (127.0 / kmax)),
                 jnp.rint(vf * (127.0 / vmax))], axis=1
            ).astype(jnp.int8)
            scsend_ref[j] = (
                jnp.concatenate([kmax, vmax], axis=1) * (1.0 / 127.0)
            ).astype(jnp.bfloat16)

        pl.semaphore_wait(barrier_sem, 2)

        rdma_s = pltpu.make_async_remote_copy(
            src_ref=scsend_ref, dst_ref=psc_ref,
            send_sem=s1s, recv_sem=r1s,
            device_id=y_nbr, device_id_type=pl.DeviceIdType.MESH,
        )
        rdma_s.start()
        rdma_kv = []
        for c in range(C):
            dkv = pltpu.make_async_remote_copy(
                src_ref=kv8send_ref.at[pl.ds(c * B, B)],
                dst_ref=pkv8_ref.at[pl.ds(c * B, B)],
                send_sem=s1.at[c], recv_sem=r1.at[c],
                device_id=y_nbr, device_id_type=pl.DeviceIdType.MESH,
            )
            dkv.start()
            rdma_kv.append(dkv)

        ones_col = jnp.ones((SEQ, 1), jnp.bfloat16)

        def qk_exp(i, k):
            s = lax.dot_general(
                q_ref[i], k, (((1,), (1,)), ((), ())),
                preferred_element_type=jnp.float32,
            )
            return jnp.exp2(s).astype(jnp.bfloat16)

        def local_attend(j):
            i = base + j
            e = qk_exp(i, k_ref[i])
            vaug = jnp.concatenate([v_ref[i], ones_col], axis=1)
            acc_ref[j] = lax.dot_general(
                e, vaug, (((1,), (0,)), ((), ())),
                preferred_element_type=jnp.float32,
            )

        def peer_attend_finalize(j):
            sc = psc_ref[j]
            kp = pkv8_ref[j][:, :D].astype(jnp.bfloat16) * sc[:, 0:1]
            vp = pkv8_ref[j][:, D:].astype(jnp.bfloat16) * sc[:, 1:2]
            e = qk_exp(base + j, kp)
            vaug = jnp.concatenate([vp, ones_col], axis=1)
            pv = lax.dot_general(
                e, vaug, (((1,), (0,)), ((), ())),
                preferred_element_type=jnp.float32,
            )
            tot = acc_ref[j] + pv
            o = tot[:, :D] * (1.0 / tot[:, D:D + 1])
            out_ref[base + j] = o
            myout_ref[j] = o.astype(jnp.bfloat16)

        rdma_o = []
        for c in range(C):
            for j in range(c * B, (c + 1) * B):
                local_attend(j)
            if c == 0:
                rdma_s.wait_recv()
            rdma_kv[c].wait_recv()
            for j in range(c * B, (c + 1) * B):
                peer_attend_finalize(j)
            do = pltpu.make_async_remote_copy(
                src_ref=myout_ref.at[pl.ds(c * B, B)],
                dst_ref=outx_ref.at[pl.ds(c * B, B)],
                send_sem=s3.at[c], recv_sem=r3.at[c],
                device_id=x_nbr, device_id_type=pl.DeviceIdType.MESH,
            )
            do.start()
            rdma_o.append(do)

        other = (1 - my_x) * HALF
        for c in range(C):
            rdma_o[c].wait_recv()
            for j in range(c * B, (c + 1) * B):
                out_ref[other + j] = outx_ref[j].astype(jnp.float32)

        rdma_s.wait_send()
        for c in range(C):
            rdma_kv[c].wait_send()
            rdma_o[c].wait_send()

    out = pl.pallas_call(
        body,
        out_shape=jax.ShapeDtypeStruct((BH, SEQ, D), jnp.float32),
        in_specs=[pl.BlockSpec(memory_space=pltpu.VMEM)] * 3,
        out_specs=pl.BlockSpec(memory_space=pltpu.VMEM),
        scratch_shapes=[
            pltpu.VMEM((HALF, SEQ, 2 * D), jnp.int8),
            pltpu.VMEM((HALF, SEQ, 2), jnp.bfloat16),
            pltpu.VMEM((HALF, SEQ, 2 * D), jnp.int8),
            pltpu.VMEM((HALF, SEQ, 2), jnp.bfloat16),
            pltpu.VMEM((HALF, SEQ, D + 1), jnp.float32),
            pltpu.VMEM((HALF, SEQ, D), jnp.bfloat16),
            pltpu.VMEM((HALF, SEQ, D), jnp.bfloat16),
            pltpu.SemaphoreType.DMA((C,)),
            pltpu.SemaphoreType.DMA((C,)),
            pltpu.SemaphoreType.DMA,
            pltpu.SemaphoreType.DMA,
            pltpu.SemaphoreType.DMA((C,)),
            pltpu.SemaphoreType.DMA((C,)),
        ],
        compiler_params=pltpu.CompilerParams(collective_id=0),
    )(Qt, Kb, Vb)

    return out.reshape(2, 8, SEQ, D).transpose(0, 2, 1, 3)
